# baseline (speedup 1.0000x reference)
import functools

import jax
import jax.numpy as jnp
from jax import lax
from jax.experimental import pallas as pl
from jax.experimental.pallas import tpu as pltpu, tpu_sc as plsc

N = 10000
E = 320000
NF = 128
NC = 40
HC = 64
H = 8
NCP = 48

_SC_INFO = plsc.get_sparse_core_info()
SC_CORES = _SC_INFO.num_cores
SC_SUBCORES = _SC_INFO.num_subcores
NW = SC_CORES * SC_SUBCORES
EPT = E // NW
BB = 80
NBATCH = EPT // BB
NPAD = 10240
RPS = NPAD // SC_SUBCORES

_mesh = plsc.VectorSubcoreMesh(core_axis_name="c", subcore_axis_name="s")


_TCB = 1000


def _tc1_body(x_ref, w1_ref, asv_ref, adv_ref, h_ref, as_ref, ad_ref):
    h = jnp.dot(x_ref[...], w1_ref[...], preferred_element_type=jnp.float32)
    h_ref[...] = h
    k = lax.broadcasted_iota(jnp.int32, (H * HC, H), 0) // HC
    g = (k == lax.broadcasted_iota(jnp.int32, (H * HC, H), 1)).astype(jnp.float32)
    as_ref[...] = jnp.dot(h * asv_ref[...], g, preferred_element_type=jnp.float32)
    ad_ref[...] = jnp.dot(h * adv_ref[...], g, preferred_element_type=jnp.float32)


def _tc1(x, W1, asv, adv):
    return pl.pallas_call(
        _tc1_body,
        grid=(N // _TCB,),
        in_specs=[
            pl.BlockSpec((_TCB, NF), lambda i: (i, 0)),
            pl.BlockSpec((NF, H * HC), lambda i: (0, 0)),
            pl.BlockSpec((1, H * HC), lambda i: (0, 0)),
            pl.BlockSpec((1, H * HC), lambda i: (0, 0)),
        ],
        out_specs=[
            pl.BlockSpec((_TCB, H * HC), lambda i: (i, 0)),
            pl.BlockSpec((_TCB, H), lambda i: (i, 0)),
            pl.BlockSpec((_TCB, H), lambda i: (i, 0)),
        ],
        out_shape=[
            jax.ShapeDtypeStruct((N, H * HC), jnp.float32),
            jax.ShapeDtypeStruct((N, H), jnp.float32),
            jax.ShapeDtypeStruct((N, H), jnp.float32),
        ],
    )(x, W1, asv, adv)


def _tc2_body(a00, a01, a10, a11, a20, a21, a30, a31,
              d00_ref, d01_ref, d10_ref, d11_ref, b1_ref,
              w2_ref, a2s_ref, a2d_ref, emb_ref, h2_ref, as2_ref, ad2_ref):
    acc = jnp.concatenate([a00[...] + a01[...], a10[...] + a11[...],
                           a20[...] + a21[...], a30[...] + a31[...]], axis=1)
    den = jnp.concatenate([d00_ref[...] + d01_ref[...],
                           d10_ref[...] + d11_ref[...]], axis=1)
    k = lax.broadcasted_iota(jnp.int32, (H, H * HC), 1) // HC
    gt = (k == lax.broadcasted_iota(jnp.int32, (H, H * HC), 0)).astype(jnp.float32)
    denb = jnp.dot(den, gt, preferred_element_type=jnp.float32)
    emb = acc / (denb + 1e-16) + b1_ref[...]
    emb_ref[...] = emb
    hact = jnp.where(emb > 0, emb, jnp.exp(jnp.minimum(emb, 0.0)) - 1.0)
    h2 = jnp.dot(hact, w2_ref[...], preferred_element_type=jnp.float32)
    h2_ref[...] = h2
    as2_ref[...] = jnp.dot(h2, a2s_ref[...], preferred_element_type=jnp.float32)
    ad2_ref[...] = jnp.dot(h2, a2d_ref[...], preferred_element_type=jnp.float32)


def _tc2(accs, d00, d01, d10, d11, b1, W2p, A2s, A2d):
    hh = H // 2
    return pl.pallas_call(
        _tc2_body,
        grid=(N // _TCB,),
        in_specs=[
            pl.BlockSpec((_TCB, 128), lambda i: (i, 0)),
            pl.BlockSpec((_TCB, 128), lambda i: (i, 0)),
            pl.BlockSpec((_TCB, 128), lambda i: (i, 0)),
            pl.BlockSpec((_TCB, 128), lambda i: (i, 0)),
            pl.BlockSpec((_TCB, 128), lambda i: (i, 0)),
            pl.BlockSpec((_TCB, 128), lambda i: (i, 0)),
            pl.BlockSpec((_TCB, 128), lambda i: (i, 0)),
            pl.BlockSpec((_TCB, 128), lambda i: (i, 0)),
            pl.BlockSpec((_TCB, hh), lambda i: (i, 0)),
            pl.BlockSpec((_TCB, hh), lambda i: (i, 0)),
            pl.BlockSpec((_TCB, hh), lambda i: (i, 0)),
            pl.BlockSpec((_TCB, hh), lambda i: (i, 0)),
            pl.BlockSpec((1, H * HC), lambda i: (0, 0)),
            pl.BlockSpec((H * HC, NCP), lambda i: (0, 0)),
            pl.BlockSpec((NCP, H), lambda i: (0, 0)),
            pl.BlockSpec((NCP, H), lambda i: (0, 0)),
        ],
        out_specs=[
            pl.BlockSpec((_TCB, H * HC), lambda i: (i, 0)),
            pl.BlockSpec((_TCB, NCP), lambda i: (i, 0)),
            pl.BlockSpec((_TCB, H), lambda i: (i, 0)),
            pl.BlockSpec((_TCB, H), lambda i: (i, 0)),
        ],
        out_shape=[
            jax.ShapeDtypeStruct((N, H * HC), jnp.float32),
            jax.ShapeDtypeStruct((N, NCP), jnp.float32),
            jax.ShapeDtypeStruct((N, H), jnp.float32),
            jax.ShapeDtypeStruct((N, H), jnp.float32),
        ],
    )(*accs, d00, d01, d10, d11, b1, W2p, A2s, A2d)


def _tc3_body(p0_ref, p1_ref, b2_ref, out_ref):
    acc = p0_ref[...] + p1_ref[...]
    den = acc[:, NC:NC + 1]
    out_ref[...] = acc / (den + 1e-16) + b2_ref[...]


def _tc3(p0, p1, b2p):
    return pl.pallas_call(
        _tc3_body,
        grid=(N // _TCB,),
        in_specs=[
            pl.BlockSpec((_TCB, NCP), lambda i: (i, 0)),
            pl.BlockSpec((_TCB, NCP), lambda i: (i, 0)),
            pl.BlockSpec((1, NCP), lambda i: (0, 0)),
        ],
        out_specs=pl.BlockSpec((_TCB, NCP), lambda i: (i, 0)),
        out_shape=jax.ShapeDtypeStruct((N, NCP), jnp.float32),
    )(p0, p1, b2p)


def _wid():
    return lax.axis_index("s") * SC_CORES + lax.axis_index("c")


def _leaky_exp(a):
    return jnp.exp(jnp.maximum(a, 0.2 * a))


@functools.partial(
    pl.kernel, mesh=_mesh,
    compiler_params=pltpu.CompilerParams(needs_layout_passes=False),
    out_type=(jax.ShapeDtypeStruct((E * 4,), jnp.float32),
              jax.ShapeDtypeStruct((E * 4,), jnp.float32)),
    scratch_types=[
        pltpu.VMEM((N * 4,), jnp.float32),
        pltpu.VMEM((N * 4,), jnp.float32),
        pltpu.VMEM((EPT,), jnp.int32),
        pltpu.VMEM((EPT,), jnp.int32),
        pltpu.VMEM((BB * 4,), jnp.float32),
    ],
)
def _sck1(src_hbm, dst_hbm, aslo, ashi, adlo, adhi, wlo_hbm, whi_hbm,
          tabs, tabd, srcv, dstv, wbuf):
    base = _wid() * EPT
    pltpu.sync_copy(src_hbm.at[pl.ds(base, EPT)], srcv)
    pltpu.sync_copy(dst_hbm.at[pl.ds(base, EPT)], dstv)
    for half in range(2):
        pltpu.sync_copy(aslo if half == 0 else ashi, tabs)
        pltpu.sync_copy(adlo if half == 0 else adhi, tabd)
        w_hbm = wlo_hbm if half == 0 else whi_hbm

        def bbody(b, carry):
            off = b * BB
            for g in range(BB // 16):
                s16 = srcv[pl.ds(off + g * 16, 16)] * 4
                d16 = dstv[pl.ds(off + g * 16, 16)] * 4
                lanes = lax.iota(jnp.int32, 16) * 4 + g * 64
                for h in range(4):
                    a = plsc.load_gather(tabs, [s16 + h]) + plsc.load_gather(tabd, [d16 + h])
                    plsc.store_scatter(wbuf, [lanes + h], _leaky_exp(a))
            pltpu.sync_copy(wbuf, w_hbm.at[pl.ds((base + off) * 4, BB * 4)])
            return carry

        lax.fori_loop(0, NBATCH, bbody, 0)


@functools.partial(
    pl.kernel, mesh=_mesh,
    compiler_params=pltpu.CompilerParams(needs_layout_passes=False),
    out_type=(
        jax.ShapeDtypeStruct((SC_CORES, NPAD, 128), jnp.float32),
        jax.ShapeDtypeStruct((SC_CORES, NPAD, 128), jnp.float32),
        jax.ShapeDtypeStruct((SC_CORES, NPAD, 128), jnp.float32),
        jax.ShapeDtypeStruct((SC_CORES, NPAD, 128), jnp.float32),
        jax.ShapeDtypeStruct((2, SC_CORES, NPAD, 4), jnp.float32),
    ),
    scratch_types=[
        pltpu.VMEM((BB,), jnp.int32),
        pltpu.VMEM((BB,), jnp.int32),
        pltpu.VMEM((BB,), jnp.int32),
        pltpu.VMEM((BB, 128), jnp.float32),
        pltpu.VMEM((BB, 128), jnp.float32),
        pltpu.VMEM((BB * 4,), jnp.float32),
        pltpu.VMEM((BB, 4), jnp.float32),
        pltpu.VMEM((BB, 128), jnp.float32),
        pltpu.VMEM_SHARED((NPAD, 128), jnp.float32),
        pltpu.VMEM_SHARED((NPAD, 4), jnp.float32),
        pltpu.SemaphoreType.DMA,
    ],
)
def _sck2(src_hbm, dst_hbm, h1c_hbm, wlo, whi, wrlo, wrhi, z128, z4,
          acc0_out, acc1_out, acc2_out, acc3_out, den_out,
          sbatch, gidx, didx, rows, msg, wrow, wrow4, bounce, acc_sp, den_sp,
          sem):
    cid = lax.axis_index("c")
    sid = lax.axis_index("s")
    base = _wid() * EPT
    accs = (acc0_out, acc1_out, acc2_out, acc3_out)
    for c in range(4):
        hh0 = (2 * c) % 4
        w_hbm = wlo if c < 2 else whi
        wr_hbm = wrlo if c < 2 else wrhi
        @pl.when(sid == 0)
        def _():
            pltpu.sync_copy(z128, acc_sp)
        if c in (0, 2):
            @pl.when(sid == 0)
            def _():
                pltpu.sync_copy(z4, den_sp)
        plsc.subcore_barrier()

        def bbody(b, carry):
            off = b * BB
            pltpu.sync_copy(src_hbm.at[pl.ds(base + off, BB)], sbatch)
            pltpu.sync_copy(dst_hbm.at[pl.ds(base + off, BB)], didx)
            for g in range(BB // 16):
                gidx[pl.ds(g * 16, 16)] = sbatch[pl.ds(g * 16, 16)] + c * N
            pltpu.async_copy(h1c_hbm.at[gidx], rows, sem).wait()
            pltpu.sync_copy(w_hbm.at[pl.ds((base + off) * 4, BB * 4)], wrow)

            def ebody(q, cc):
                w16 = wrow[pl.ds(q * 16, 16)]
                for j in range(4):
                    e = q * 4 + j
                    s0 = lax.broadcast_in_dim(w16[4 * j + hh0], (16,), ())
                    s1 = lax.broadcast_in_dim(w16[4 * j + hh0 + 1], (16,), ())
                    for k in range(8):
                        sk = s0 if k < 4 else s1
                        msg[e, pl.ds(k * 16, 16)] = rows[e, pl.ds(k * 16, 16)] * sk
                return cc

            lax.fori_loop(0, BB // 4, ebody, 0)
            pltpu.sync_copy(msg, acc_sp.at[didx], add=True)
            if c in (0, 2):
                pltpu.sync_copy(wr_hbm.at[pl.ds(base + off, BB)], wrow4)
                pltpu.sync_copy(wrow4, den_sp.at[didx], add=True)
            return carry

        lax.fori_loop(0, NBATCH, bbody, 0)
        plsc.subcore_barrier()

        @pl.when(sid == 0)
        def _():
            pltpu.sync_copy(acc_sp, accs[c].at[cid])
        if c in (0, 2):
            @pl.when(sid == 0)
            def _():
                pltpu.sync_copy(den_sp, den_out.at[c // 2, cid])
        plsc.subcore_barrier()


@functools.partial(
    pl.kernel, mesh=_mesh,
    compiler_params=pltpu.CompilerParams(needs_layout_passes=False),
    out_type=jax.ShapeDtypeStruct((E,), jnp.float32),
    scratch_types=[
        pltpu.VMEM((N,), jnp.float32),
        pltpu.VMEM((N,), jnp.float32),
        pltpu.VMEM((EPT,), jnp.int32),
        pltpu.VMEM((EPT,), jnp.int32),
        pltpu.VMEM((BB,), jnp.float32),
    ],
)
def _sck3(src_hbm, dst_hbm, as2, ad2, w_hbm, tabs, tabd, srcv, dstv, wbuf):
    base = _wid() * EPT
    pltpu.sync_copy(src_hbm.at[pl.ds(base, EPT)], srcv)
    pltpu.sync_copy(dst_hbm.at[pl.ds(base, EPT)], dstv)
    pltpu.sync_copy(as2, tabs)
    pltpu.sync_copy(ad2, tabd)

    def bbody(b, carry):
        off = b * BB
        for g in range(BB // 16):
            s16 = srcv[pl.ds(off + g * 16, 16)]
            d16 = dstv[pl.ds(off + g * 16, 16)]
            a = plsc.load_gather(tabs, [s16]) + plsc.load_gather(tabd, [d16])
            wbuf[pl.ds(g * 16, 16)] = _leaky_exp(a)
        pltpu.sync_copy(wbuf, w_hbm.at[pl.ds(base + off, BB)])
        return carry

    lax.fori_loop(0, NBATCH, bbody, 0)


@functools.partial(
    pl.kernel, mesh=_mesh,
    compiler_params=pltpu.CompilerParams(needs_layout_passes=False,
                                         use_tc_tiling_on_sc=False),
    out_type=jax.ShapeDtypeStruct((SC_CORES, NPAD, NCP), jnp.float32),
    scratch_types=[
        pltpu.VMEM((EPT,), jnp.int32),
        pltpu.VMEM((EPT,), jnp.int32),
        pltpu.VMEM((BB,), jnp.int32),
        pltpu.VMEM((BB, NCP), jnp.float32),
        pltpu.VMEM((BB, NCP), jnp.float32),
        pltpu.VMEM((BB,), jnp.float32),
        pltpu.VMEM((BB, NCP), jnp.float32),
        pltpu.VMEM_SHARED((NPAD, NCP), jnp.float32),
        pltpu.SemaphoreType.DMA,
    ],
)
def _sck4(src_hbm, dst_hbm, h2_hbm, w_hbm, z48, acc_out,
          srcv, dstv, didx, rows, msg, wv, bounce, acc_sp, sem):
    cid = lax.axis_index("c")
    sid = lax.axis_index("s")
    base = _wid() * EPT
    pltpu.sync_copy(src_hbm.at[pl.ds(base, EPT)], srcv)
    pltpu.sync_copy(dst_hbm.at[pl.ds(base, EPT)], dstv)
    @pl.when(sid == 0)
    def _():
        pltpu.sync_copy(z48, acc_sp)
    plsc.subcore_barrier()
    denlane = (lax.iota(jnp.int32, 16) == (NC - 32)).astype(jnp.float32)

    def bbody(b, carry):
        off = b * BB
        for g in range(BB // 16):
            didx[pl.ds(g * 16, 16)] = dstv[pl.ds(off + g * 16, 16)]
        pltpu.async_copy(h2_hbm.at[srcv.at[pl.ds(off, BB)]], rows, sem).wait()
        pltpu.sync_copy(w_hbm.at[pl.ds(base + off, BB)], wv)

        def ebody(g, cc):
            w16 = wv[pl.ds(g * 16, 16)]
            for i in range(16):
                e = g * 16 + i
                s = lax.broadcast_in_dim(w16[i], (16,), ())
                msg[e, pl.ds(0, 16)] = rows[e, pl.ds(0, 16)] * s
                msg[e, pl.ds(16, 16)] = rows[e, pl.ds(16, 16)] * s
                msg[e, pl.ds(32, 16)] = rows[e, pl.ds(32, 16)] * s + denlane * s
            return cc

        lax.fori_loop(0, BB // 16, ebody, 0)
        pltpu.sync_copy(msg, acc_sp.at[didx], add=True)
        return carry

    lax.fori_loop(0, NBATCH, bbody, 0)
    plsc.subcore_barrier()

    @pl.when(sid == 0)
    def _():
        pltpu.sync_copy(acc_sp, acc_out.at[cid])


_DEBUG_STAGE = 2


def _pad_rows(a):
    return jnp.pad(a, ((0, NPAD - N),) + ((0, 0),) * (a.ndim - 1))


def kernel(x, edge_index, W1, a_src1, a_dst1, b1, W2, a_src2, a_dst2, b2):
    src = edge_index[0]
    dst = edge_index[1]
    asv = a_src1.reshape(1, H * HC)
    adv = a_dst1.reshape(1, H * HC)

    h1, as1, ad1 = _tc1(x, W1, asv, adv)
    h1c = h1.reshape(N, 4, 128).transpose(1, 0, 2).reshape(4 * N, 128)

    aslo = as1[:, :4].reshape(-1)
    ashi = as1[:, 4:].reshape(-1)
    adlo = ad1[:, :4].reshape(-1)
    adhi = ad1[:, 4:].reshape(-1)
    w1lo, w1hi = _sck1(src, dst, aslo, ashi, adlo, adhi)
    w1rlo = w1lo.reshape(E, 4)
    w1rhi = w1hi.reshape(E, 4)

    w = jnp.concatenate([w1rlo, w1rhi], axis=1)
    msg = h1.reshape(N, H, HC)[src] * w[..., None]
    accx = jax.ops.segment_sum(msg, dst, num_segments=N).reshape(N, H * HC)
    denx = jax.ops.segment_sum(w, dst, num_segments=N)
    zc = jnp.zeros((N, 128), jnp.float32)
    accs = [accx[:, 0:128], zc, accx[:, 128:256], zc,
            accx[:, 256:384], zc, accx[:, 384:512], zc]
    zz = jnp.zeros((NPAD, 4), jnp.float32)
    den1 = jnp.stack([jnp.stack([_pad_rows(denx[:, :4]), zz]),
                      jnp.stack([_pad_rows(denx[:, 4:]), zz])])

    W2p = jnp.pad(W2, ((0, 0), (0, NCP - NC)))
    a2sv = jnp.pad(a_src2.reshape(NC), (0, NCP - NC))
    a2dv = jnp.pad(a_dst2.reshape(NC), (0, NCP - NC))
    A2s = jnp.tile(a2sv[:, None], (1, H))
    A2d = jnp.tile(a2dv[:, None], (1, H))
    emb, h2p, as2, ad2 = _tc2(accs,
                              den1[0, 0, :N], den1[0, 1, :N],
                              den1[1, 0, :N], den1[1, 1, :N],
                              b1.reshape(1, H * HC), W2p, A2s, A2d)

    if _DEBUG_STAGE >= 3:
        w2 = _sck3(src, dst, as2[:, 0], ad2[:, 0])
    else:
        a = as2[src, 0] + ad2[dst, 0]
        w2 = jnp.exp(jnp.where(a > 0, a, 0.2 * a))

    z48 = jnp.zeros((NPAD, NCP), jnp.float32)
    if _DEBUG_STAGE >= 4:
        acc2 = _sck4(src, dst, h2p, w2, z48)
    else:
        msg2 = h2p[src] * w2[:, None]
        accx2 = jax.ops.segment_sum(msg2, dst, num_segments=N)
        den2 = jax.ops.segment_sum(w2, dst, num_segments=N)
        accx2 = accx2.at[:, NC].set(den2)
        acc2 = jnp.stack([_pad_rows(accx2), jnp.zeros((NPAD, NCP), jnp.float32)])

    b2p = jnp.pad(b2, (0, NCP - NC)).reshape(1, NCP)
    out48 = _tc3(acc2[0, :N], acc2[1, :N], b2p)
    logits = out48[:, :NC]
    return (logits, emb)


# speedup vs baseline: 1.0001x; 1.0001x over previous
import functools

import jax
import jax.numpy as jnp
from jax import lax
from jax.experimental import pallas as pl
from jax.experimental.pallas import tpu as pltpu, tpu_sc as plsc

N = 10000
E = 320000
NF = 128
NC = 40
HC = 64
H = 8
NCP = 48

_SC_INFO = plsc.get_sparse_core_info()
SC_CORES = _SC_INFO.num_cores
SC_SUBCORES = _SC_INFO.num_subcores
NW = SC_CORES * SC_SUBCORES
EPT = E // NW
BB = 80
NBATCH = EPT // BB
NPAD = 10240
RPS = NPAD // SC_SUBCORES

_mesh = plsc.VectorSubcoreMesh(core_axis_name="c", subcore_axis_name="s")


_TCB = 1000


def _tc1_body(x_ref, w1_ref, asv_ref, adv_ref, h_ref, as_ref, ad_ref):
    h = jnp.dot(x_ref[...], w1_ref[...], preferred_element_type=jnp.float32)
    h_ref[...] = h
    k = lax.broadcasted_iota(jnp.int32, (H * HC, H), 0) // HC
    g = (k == lax.broadcasted_iota(jnp.int32, (H * HC, H), 1)).astype(jnp.float32)
    as_ref[...] = jnp.dot(h * asv_ref[...], g, preferred_element_type=jnp.float32)
    ad_ref[...] = jnp.dot(h * adv_ref[...], g, preferred_element_type=jnp.float32)


def _tc1(x, W1, asv, adv):
    return pl.pallas_call(
        _tc1_body,
        grid=(N // _TCB,),
        in_specs=[
            pl.BlockSpec((_TCB, NF), lambda i: (i, 0)),
            pl.BlockSpec((NF, H * HC), lambda i: (0, 0)),
            pl.BlockSpec((1, H * HC), lambda i: (0, 0)),
            pl.BlockSpec((1, H * HC), lambda i: (0, 0)),
        ],
        out_specs=[
            pl.BlockSpec((_TCB, H * HC), lambda i: (i, 0)),
            pl.BlockSpec((_TCB, H), lambda i: (i, 0)),
            pl.BlockSpec((_TCB, H), lambda i: (i, 0)),
        ],
        out_shape=[
            jax.ShapeDtypeStruct((N, H * HC), jnp.float32),
            jax.ShapeDtypeStruct((N, H), jnp.float32),
            jax.ShapeDtypeStruct((N, H), jnp.float32),
        ],
    )(x, W1, asv, adv)


def _tc2_body(a00, a01, a10, a11, a20, a21, a30, a31,
              d00_ref, d01_ref, d10_ref, d11_ref, b1_ref,
              w2_ref, a2s_ref, a2d_ref, emb_ref, h2_ref, as2_ref, ad2_ref):
    acc = jnp.concatenate([a00[...] + a01[...], a10[...] + a11[...],
                           a20[...] + a21[...], a30[...] + a31[...]], axis=1)
    den = jnp.concatenate([d00_ref[...] + d01_ref[...],
                           d10_ref[...] + d11_ref[...]], axis=1)
    k = lax.broadcasted_iota(jnp.int32, (H, H * HC), 1) // HC
    gt = (k == lax.broadcasted_iota(jnp.int32, (H, H * HC), 0)).astype(jnp.float32)
    denb = jnp.dot(den, gt, preferred_element_type=jnp.float32)
    emb = acc / (denb + 1e-16) + b1_ref[...]
    emb_ref[...] = emb
    hact = jnp.where(emb > 0, emb, jnp.exp(jnp.minimum(emb, 0.0)) - 1.0)
    h2 = jnp.dot(hact, w2_ref[...], preferred_element_type=jnp.float32)
    h2_ref[...] = h2
    as2_ref[...] = jnp.dot(h2, a2s_ref[...], preferred_element_type=jnp.float32)
    ad2_ref[...] = jnp.dot(h2, a2d_ref[...], preferred_element_type=jnp.float32)


def _tc2(accs, d00, d01, d10, d11, b1, W2p, A2s, A2d):
    hh = H // 2
    return pl.pallas_call(
        _tc2_body,
        grid=(N // _TCB,),
        in_specs=[
            pl.BlockSpec((_TCB, 128), lambda i: (i, 0)),
            pl.BlockSpec((_TCB, 128), lambda i: (i, 0)),
            pl.BlockSpec((_TCB, 128), lambda i: (i, 0)),
            pl.BlockSpec((_TCB, 128), lambda i: (i, 0)),
            pl.BlockSpec((_TCB, 128), lambda i: (i, 0)),
            pl.BlockSpec((_TCB, 128), lambda i: (i, 0)),
            pl.BlockSpec((_TCB, 128), lambda i: (i, 0)),
            pl.BlockSpec((_TCB, 128), lambda i: (i, 0)),
            pl.BlockSpec((_TCB, hh), lambda i: (i, 0)),
            pl.BlockSpec((_TCB, hh), lambda i: (i, 0)),
            pl.BlockSpec((_TCB, hh), lambda i: (i, 0)),
            pl.BlockSpec((_TCB, hh), lambda i: (i, 0)),
            pl.BlockSpec((1, H * HC), lambda i: (0, 0)),
            pl.BlockSpec((H * HC, NCP), lambda i: (0, 0)),
            pl.BlockSpec((NCP, H), lambda i: (0, 0)),
            pl.BlockSpec((NCP, H), lambda i: (0, 0)),
        ],
        out_specs=[
            pl.BlockSpec((_TCB, H * HC), lambda i: (i, 0)),
            pl.BlockSpec((_TCB, NCP), lambda i: (i, 0)),
            pl.BlockSpec((_TCB, H), lambda i: (i, 0)),
            pl.BlockSpec((_TCB, H), lambda i: (i, 0)),
        ],
        out_shape=[
            jax.ShapeDtypeStruct((N, H * HC), jnp.float32),
            jax.ShapeDtypeStruct((N, NCP), jnp.float32),
            jax.ShapeDtypeStruct((N, H), jnp.float32),
            jax.ShapeDtypeStruct((N, H), jnp.float32),
        ],
    )(*accs, d00, d01, d10, d11, b1, W2p, A2s, A2d)


def _tc3_body(p0_ref, p1_ref, b2_ref, out_ref):
    acc = p0_ref[...] + p1_ref[...]
    den = acc[:, NC:NC + 1]
    out_ref[...] = acc / (den + 1e-16) + b2_ref[...]


def _tc3(p0, p1, b2p):
    return pl.pallas_call(
        _tc3_body,
        grid=(N // _TCB,),
        in_specs=[
            pl.BlockSpec((_TCB, NCP), lambda i: (i, 0)),
            pl.BlockSpec((_TCB, NCP), lambda i: (i, 0)),
            pl.BlockSpec((1, NCP), lambda i: (0, 0)),
        ],
        out_specs=pl.BlockSpec((_TCB, NCP), lambda i: (i, 0)),
        out_shape=jax.ShapeDtypeStruct((N, NCP), jnp.float32),
    )(p0, p1, b2p)


def _wid():
    return lax.axis_index("s") * SC_CORES + lax.axis_index("c")


def _leaky_exp(a):
    return jnp.exp(jnp.maximum(a, 0.2 * a))


@functools.partial(
    pl.kernel, mesh=_mesh,
    compiler_params=pltpu.CompilerParams(needs_layout_passes=False),
    out_type=(jax.ShapeDtypeStruct((E * 4,), jnp.float32),
              jax.ShapeDtypeStruct((E * 4,), jnp.float32)),
    scratch_types=[
        pltpu.VMEM((N * 4,), jnp.float32),
        pltpu.VMEM((N * 4,), jnp.float32),
        pltpu.VMEM((EPT,), jnp.int32),
        pltpu.VMEM((EPT,), jnp.int32),
        pltpu.VMEM((BB * 4,), jnp.float32),
    ],
)
def _sck1(src_hbm, dst_hbm, aslo, ashi, adlo, adhi, wlo_hbm, whi_hbm,
          tabs, tabd, srcv, dstv, wbuf):
    base = _wid() * EPT
    pltpu.sync_copy(src_hbm.at[pl.ds(base, EPT)], srcv)
    pltpu.sync_copy(dst_hbm.at[pl.ds(base, EPT)], dstv)
    for half in range(2):
        pltpu.sync_copy(aslo if half == 0 else ashi, tabs)
        pltpu.sync_copy(adlo if half == 0 else adhi, tabd)
        w_hbm = wlo_hbm if half == 0 else whi_hbm

        def bbody(b, carry):
            off = b * BB
            for g in range(BB // 16):
                s16 = srcv[pl.ds(off + g * 16, 16)] * 4
                d16 = dstv[pl.ds(off + g * 16, 16)] * 4
                lanes = lax.iota(jnp.int32, 16) * 4 + g * 64
                for h in range(4):
                    a = plsc.load_gather(tabs, [s16 + h]) + plsc.load_gather(tabd, [d16 + h])
                    plsc.store_scatter(wbuf, [lanes + h], _leaky_exp(a))
            pltpu.sync_copy(wbuf, w_hbm.at[pl.ds((base + off) * 4, BB * 4)])
            return carry

        lax.fori_loop(0, NBATCH, bbody, 0)


@functools.partial(
    pl.kernel, mesh=_mesh,
    compiler_params=pltpu.CompilerParams(needs_layout_passes=False),
    out_type=jax.ShapeDtypeStruct((E,), jnp.float32),
    scratch_types=[
        pltpu.VMEM((N,), jnp.float32),
        pltpu.VMEM((N,), jnp.float32),
        pltpu.VMEM((EPT,), jnp.int32),
        pltpu.VMEM((EPT,), jnp.int32),
        pltpu.VMEM((BB,), jnp.float32),
    ],
)
def _sck3(src_hbm, dst_hbm, as2, ad2, w_hbm, tabs, tabd, srcv, dstv, wbuf):
    base = _wid() * EPT
    pltpu.sync_copy(src_hbm.at[pl.ds(base, EPT)], srcv)
    pltpu.sync_copy(dst_hbm.at[pl.ds(base, EPT)], dstv)
    pltpu.sync_copy(as2, tabs)
    pltpu.sync_copy(ad2, tabd)

    def bbody(b, carry):
        off = b * BB
        for g in range(BB // 16):
            s16 = srcv[pl.ds(off + g * 16, 16)]
            d16 = dstv[pl.ds(off + g * 16, 16)]
            a = plsc.load_gather(tabs, [s16]) + plsc.load_gather(tabd, [d16])
            wbuf[pl.ds(g * 16, 16)] = _leaky_exp(a)
        pltpu.sync_copy(wbuf, w_hbm.at[pl.ds(base + off, BB)])
        return carry

    lax.fori_loop(0, NBATCH, bbody, 0)


_DEBUG_STAGE = 2


def _pad_rows(a):
    return jnp.pad(a, ((0, NPAD - N),) + ((0, 0),) * (a.ndim - 1))


def kernel(x, edge_index, W1, a_src1, a_dst1, b1, W2, a_src2, a_dst2, b2):
    src = edge_index[0]
    dst = edge_index[1]
    asv = a_src1.reshape(1, H * HC)
    adv = a_dst1.reshape(1, H * HC)

    h1, as1, ad1 = _tc1(x, W1, asv, adv)

    aslo = as1[:, :4].reshape(-1)
    ashi = as1[:, 4:].reshape(-1)
    adlo = ad1[:, :4].reshape(-1)
    adhi = ad1[:, 4:].reshape(-1)
    w1lo, w1hi = _sck1(src, dst, aslo, ashi, adlo, adhi)
    w1rlo = w1lo.reshape(E, 4)
    w1rhi = w1hi.reshape(E, 4)

    w = jnp.concatenate([w1rlo, w1rhi], axis=1)
    msg = h1.reshape(N, H, HC)[src] * w[..., None]
    accx = jax.ops.segment_sum(msg, dst, num_segments=N).reshape(N, H * HC)
    denx = jax.ops.segment_sum(w, dst, num_segments=N)
    zc = jnp.zeros((N, 128), jnp.float32)
    accs = [accx[:, 0:128], zc, accx[:, 128:256], zc,
            accx[:, 256:384], zc, accx[:, 384:512], zc]
    zz = jnp.zeros((NPAD, 4), jnp.float32)
    den1 = jnp.stack([jnp.stack([_pad_rows(denx[:, :4]), zz]),
                      jnp.stack([_pad_rows(denx[:, 4:]), zz])])

    W2p = jnp.pad(W2, ((0, 0), (0, NCP - NC)))
    a2sv = jnp.pad(a_src2.reshape(NC), (0, NCP - NC))
    a2dv = jnp.pad(a_dst2.reshape(NC), (0, NCP - NC))
    A2s = jnp.tile(a2sv[:, None], (1, H))
    A2d = jnp.tile(a2dv[:, None], (1, H))
    emb, h2p, as2, ad2 = _tc2(accs,
                              den1[0, 0, :N], den1[0, 1, :N],
                              den1[1, 0, :N], den1[1, 1, :N],
                              b1.reshape(1, H * HC), W2p, A2s, A2d)

    if _DEBUG_STAGE >= 3:
        w2 = _sck3(src, dst, as2[:, 0], ad2[:, 0])
    else:
        a = as2[src, 0] + ad2[dst, 0]
        w2 = jnp.exp(jnp.where(a > 0, a, 0.2 * a))

    z48 = jnp.zeros((NPAD, NCP), jnp.float32)
    if _DEBUG_STAGE >= 4:
        acc2 = _sck4(src, dst, h2p, w2, z48)
    else:
        msg2 = h2p[src] * w2[:, None]
        accx2 = jax.ops.segment_sum(msg2, dst, num_segments=N)
        den2 = jax.ops.segment_sum(w2, dst, num_segments=N)
        accx2 = accx2.at[:, NC].set(den2)
        acc2 = jnp.stack([_pad_rows(accx2), jnp.zeros((NPAD, NCP), jnp.float32)])

    b2p = jnp.pad(b2, (0, NCP - NC)).reshape(1, NCP)
    out48 = _tc3(acc2[0, :N], acc2[1, :N], b2p)
    logits = out48[:, :NC]
    return (logits, emb)


# speedup vs baseline: 1.2935x; 1.2933x over previous
import functools

import jax
import jax.numpy as jnp
from jax import lax
from jax.experimental import pallas as pl
from jax.experimental.pallas import tpu as pltpu, tpu_sc as plsc

N = 10000
E = 320000
NF = 128
NC = 40
HC = 64
H = 8
NCP = 48

_SC_INFO = plsc.get_sparse_core_info()
SC_CORES = _SC_INFO.num_cores
SC_SUBCORES = _SC_INFO.num_subcores
NW = SC_CORES * SC_SUBCORES
EPT = E // NW
BB = 80
NBATCH = EPT // BB
NPAD = 10240
RPS = NPAD // SC_SUBCORES

_mesh = plsc.VectorSubcoreMesh(core_axis_name="c", subcore_axis_name="s")


_TCB = 1000


def _tc1_body(x_ref, w1_ref, asv_ref, adv_ref, h_ref, as_ref, ad_ref):
    h = jnp.dot(x_ref[...], w1_ref[...], preferred_element_type=jnp.float32)
    h_ref[...] = h
    k = lax.broadcasted_iota(jnp.int32, (H * HC, H), 0) // HC
    g = (k == lax.broadcasted_iota(jnp.int32, (H * HC, H), 1)).astype(jnp.float32)
    as_ref[...] = jnp.dot(h * asv_ref[...], g, preferred_element_type=jnp.float32)
    ad_ref[...] = jnp.dot(h * adv_ref[...], g, preferred_element_type=jnp.float32)


def _tc1(x, W1, asv, adv):
    return pl.pallas_call(
        _tc1_body,
        grid=(N // _TCB,),
        in_specs=[
            pl.BlockSpec((_TCB, NF), lambda i: (i, 0)),
            pl.BlockSpec((NF, H * HC), lambda i: (0, 0)),
            pl.BlockSpec((1, H * HC), lambda i: (0, 0)),
            pl.BlockSpec((1, H * HC), lambda i: (0, 0)),
        ],
        out_specs=[
            pl.BlockSpec((_TCB, H * HC), lambda i: (i, 0)),
            pl.BlockSpec((_TCB, H), lambda i: (i, 0)),
            pl.BlockSpec((_TCB, H), lambda i: (i, 0)),
        ],
        out_shape=[
            jax.ShapeDtypeStruct((N, H * HC), jnp.float32),
            jax.ShapeDtypeStruct((N, H), jnp.float32),
            jax.ShapeDtypeStruct((N, H), jnp.float32),
        ],
    )(x, W1, asv, adv)


def _tc2_body(a00, a01, a10, a11, a20, a21, a30, a31,
              d00_ref, d01_ref, d10_ref, d11_ref, b1_ref,
              w2_ref, a2s_ref, a2d_ref, emb_ref, h2_ref, as2_ref, ad2_ref):
    acc = jnp.concatenate([a00[...] + a01[...], a10[...] + a11[...],
                           a20[...] + a21[...], a30[...] + a31[...]], axis=1)
    den = jnp.concatenate([d00_ref[...] + d01_ref[...],
                           d10_ref[...] + d11_ref[...]], axis=1)
    k = lax.broadcasted_iota(jnp.int32, (H, H * HC), 1) // HC
    gt = (k == lax.broadcasted_iota(jnp.int32, (H, H * HC), 0)).astype(jnp.float32)
    denb = jnp.dot(den, gt, preferred_element_type=jnp.float32)
    emb = acc / (denb + 1e-16) + b1_ref[...]
    emb_ref[...] = emb
    hact = jnp.where(emb > 0, emb, jnp.exp(jnp.minimum(emb, 0.0)) - 1.0)
    h2 = jnp.dot(hact, w2_ref[...], preferred_element_type=jnp.float32)
    h2_ref[...] = h2
    as2_ref[...] = jnp.dot(h2, a2s_ref[...], preferred_element_type=jnp.float32)
    ad2_ref[...] = jnp.dot(h2, a2d_ref[...], preferred_element_type=jnp.float32)


def _tc2(accs, d00, d01, d10, d11, b1, W2p, A2s, A2d):
    hh = H // 2
    return pl.pallas_call(
        _tc2_body,
        grid=(N // _TCB,),
        in_specs=[
            pl.BlockSpec((_TCB, 128), lambda i: (i, 0)),
            pl.BlockSpec((_TCB, 128), lambda i: (i, 0)),
            pl.BlockSpec((_TCB, 128), lambda i: (i, 0)),
            pl.BlockSpec((_TCB, 128), lambda i: (i, 0)),
            pl.BlockSpec((_TCB, 128), lambda i: (i, 0)),
            pl.BlockSpec((_TCB, 128), lambda i: (i, 0)),
            pl.BlockSpec((_TCB, 128), lambda i: (i, 0)),
            pl.BlockSpec((_TCB, 128), lambda i: (i, 0)),
            pl.BlockSpec((_TCB, hh), lambda i: (i, 0)),
            pl.BlockSpec((_TCB, hh), lambda i: (i, 0)),
            pl.BlockSpec((_TCB, hh), lambda i: (i, 0)),
            pl.BlockSpec((_TCB, hh), lambda i: (i, 0)),
            pl.BlockSpec((1, H * HC), lambda i: (0, 0)),
            pl.BlockSpec((H * HC, NCP), lambda i: (0, 0)),
            pl.BlockSpec((NCP, H), lambda i: (0, 0)),
            pl.BlockSpec((NCP, H), lambda i: (0, 0)),
        ],
        out_specs=[
            pl.BlockSpec((_TCB, H * HC), lambda i: (i, 0)),
            pl.BlockSpec((_TCB, NCP), lambda i: (i, 0)),
            pl.BlockSpec((_TCB, H), lambda i: (i, 0)),
            pl.BlockSpec((_TCB, H), lambda i: (i, 0)),
        ],
        out_shape=[
            jax.ShapeDtypeStruct((N, H * HC), jnp.float32),
            jax.ShapeDtypeStruct((N, NCP), jnp.float32),
            jax.ShapeDtypeStruct((N, H), jnp.float32),
            jax.ShapeDtypeStruct((N, H), jnp.float32),
        ],
    )(*accs, d00, d01, d10, d11, b1, W2p, A2s, A2d)


def _tc3_body(p0_ref, p1_ref, b2_ref, out_ref):
    acc = p0_ref[...] + p1_ref[...]
    den = acc[:, NC:NC + 1]
    out_ref[...] = acc / (den + 1e-16) + b2_ref[...]


def _tc3(p0, p1, b2p):
    return pl.pallas_call(
        _tc3_body,
        grid=(N // _TCB,),
        in_specs=[
            pl.BlockSpec((_TCB, NCP), lambda i: (i, 0)),
            pl.BlockSpec((_TCB, NCP), lambda i: (i, 0)),
            pl.BlockSpec((1, NCP), lambda i: (0, 0)),
        ],
        out_specs=pl.BlockSpec((_TCB, NCP), lambda i: (i, 0)),
        out_shape=jax.ShapeDtypeStruct((N, NCP), jnp.float32),
    )(p0, p1, b2p)


def _wid():
    return lax.axis_index("s") * SC_CORES + lax.axis_index("c")


def _leaky_exp(a):
    return jnp.exp(jnp.maximum(a, 0.2 * a))


@functools.partial(
    pl.kernel, mesh=_mesh,
    compiler_params=pltpu.CompilerParams(needs_layout_passes=False),
    out_type=(jax.ShapeDtypeStruct((E * 4,), jnp.float32),
              jax.ShapeDtypeStruct((E * 4,), jnp.float32)),
    scratch_types=[
        pltpu.VMEM((N * 4,), jnp.float32),
        pltpu.VMEM((N * 4,), jnp.float32),
        pltpu.VMEM((EPT,), jnp.int32),
        pltpu.VMEM((EPT,), jnp.int32),
        pltpu.VMEM((BB * 4,), jnp.float32),
    ],
)
def _sck1(src_hbm, dst_hbm, aslo, ashi, adlo, adhi, wlo_hbm, whi_hbm,
          tabs, tabd, srcv, dstv, wbuf):
    base = _wid() * EPT
    pltpu.sync_copy(src_hbm.at[pl.ds(base, EPT)], srcv)
    pltpu.sync_copy(dst_hbm.at[pl.ds(base, EPT)], dstv)
    for half in range(2):
        pltpu.sync_copy(aslo if half == 0 else ashi, tabs)
        pltpu.sync_copy(adlo if half == 0 else adhi, tabd)
        w_hbm = wlo_hbm if half == 0 else whi_hbm

        def bbody(b, carry):
            off = b * BB
            for g in range(BB // 16):
                s16 = srcv[pl.ds(off + g * 16, 16)] * 4
                d16 = dstv[pl.ds(off + g * 16, 16)] * 4
                lanes = lax.iota(jnp.int32, 16) * 4 + g * 64
                for h in range(4):
                    a = plsc.load_gather(tabs, [s16 + h]) + plsc.load_gather(tabd, [d16 + h])
                    plsc.store_scatter(wbuf, [lanes + h], _leaky_exp(a))
            pltpu.sync_copy(wbuf, w_hbm.at[pl.ds((base + off) * 4, BB * 4)])
            return carry

        lax.fori_loop(0, NBATCH, bbody, 0)


@functools.partial(
    pl.kernel, mesh=_mesh,
    compiler_params=pltpu.CompilerParams(needs_layout_passes=False),
    out_type=jax.ShapeDtypeStruct((E,), jnp.float32),
    scratch_types=[
        pltpu.VMEM((N,), jnp.float32),
        pltpu.VMEM((N,), jnp.float32),
        pltpu.VMEM((EPT,), jnp.int32),
        pltpu.VMEM((EPT,), jnp.int32),
        pltpu.VMEM((BB,), jnp.float32),
    ],
)
def _sck3(src_hbm, dst_hbm, as2, ad2, w_hbm, tabs, tabd, srcv, dstv, wbuf):
    base = _wid() * EPT
    pltpu.sync_copy(src_hbm.at[pl.ds(base, EPT)], srcv)
    pltpu.sync_copy(dst_hbm.at[pl.ds(base, EPT)], dstv)
    pltpu.sync_copy(as2, tabs)
    pltpu.sync_copy(ad2, tabd)

    def bbody(b, carry):
        off = b * BB
        for g in range(BB // 16):
            s16 = srcv[pl.ds(off + g * 16, 16)]
            d16 = dstv[pl.ds(off + g * 16, 16)]
            a = plsc.load_gather(tabs, [s16]) + plsc.load_gather(tabd, [d16])
            wbuf[pl.ds(g * 16, 16)] = _leaky_exp(a)
        pltpu.sync_copy(wbuf, w_hbm.at[pl.ds(base + off, BB)])
        return carry

    lax.fori_loop(0, NBATCH, bbody, 0)


def _pad_rows(a):
    return jnp.pad(a, ((0, NPAD - N),) + ((0, 0),) * (a.ndim - 1))


def kernel(x, edge_index, W1, a_src1, a_dst1, b1, W2, a_src2, a_dst2, b2):
    src = edge_index[0]
    dst = edge_index[1]
    asv = a_src1.reshape(1, H * HC)
    adv = a_dst1.reshape(1, H * HC)

    h1, as1, ad1 = _tc1(x, W1, asv, adv)

    aslo = as1[:, :4].reshape(-1)
    ashi = as1[:, 4:].reshape(-1)
    adlo = ad1[:, :4].reshape(-1)
    adhi = ad1[:, 4:].reshape(-1)
    w1lo, w1hi = _sck1(src, dst, aslo, ashi, adlo, adhi)
    w1rlo = w1lo.reshape(E, 4)
    w1rhi = w1hi.reshape(E, 4)

    w = jnp.concatenate([w1rlo, w1rhi], axis=1)
    msg = h1.reshape(N, H, HC)[src] * w[..., None]
    accx = jax.ops.segment_sum(msg, dst, num_segments=N).reshape(N, H * HC)
    denx = jax.ops.segment_sum(w, dst, num_segments=N)
    zc = jnp.zeros((N, 128), jnp.float32)
    accs = [accx[:, 0:128], zc, accx[:, 128:256], zc,
            accx[:, 256:384], zc, accx[:, 384:512], zc]
    zz = jnp.zeros((NPAD, 4), jnp.float32)
    den1 = jnp.stack([jnp.stack([_pad_rows(denx[:, :4]), zz]),
                      jnp.stack([_pad_rows(denx[:, 4:]), zz])])

    W2p = jnp.pad(W2, ((0, 0), (0, NCP - NC)))
    a2sv = jnp.pad(a_src2.reshape(NC), (0, NCP - NC))
    a2dv = jnp.pad(a_dst2.reshape(NC), (0, NCP - NC))
    A2s = jnp.tile(a2sv[:, None], (1, H))
    A2d = jnp.tile(a2dv[:, None], (1, H))
    emb, h2p, as2, ad2 = _tc2(accs,
                              den1[0, 0, :N], den1[0, 1, :N],
                              den1[1, 0, :N], den1[1, 1, :N],
                              b1.reshape(1, H * HC), W2p, A2s, A2d)

    w2 = _sck3(src, dst, as2[:, 0], ad2[:, 0])

    msg2 = h2p[src] * w2[:, None]
    accx2 = jax.ops.segment_sum(msg2, dst, num_segments=N)
    den2 = jax.ops.segment_sum(w2, dst, num_segments=N)
    accx2 = accx2.at[:, NC].set(den2)
    acc2 = jnp.stack([_pad_rows(accx2), jnp.zeros((NPAD, NCP), jnp.float32)])

    b2p = jnp.pad(b2, (0, NCP - NC)).reshape(1, NCP)
    out48 = _tc3(acc2[0, :N], acc2[1, :N], b2p)
    logits = out48[:, :NC]
    return (logits, emb)
